# Initial kernel scaffold; baseline (speedup 1.0000x reference)
import jax
import jax.numpy as jnp
from jax.experimental import pallas as pl


def kernel(inputs, training):
    raise NotImplementedError("write your pallas kernel here")


# speedup vs baseline: 3.8210x; 3.8210x over previous
import numpy as np
import jax
import jax.numpy as jnp
from jax.experimental import pallas as pl

_B = 16384
_D = 128
_NBIN = 100
_RATE = np.float32(0.01)
_MASK23 = 0x3F800000


def _threefry2x32(ks0, ks1, x0, x1):
    ks2 = ks0 ^ ks1 ^ np.uint32(0x1BD11BDA)
    rot_a = (13, 15, 26, 6)
    rot_b = (17, 29, 16, 24)

    def rounds(x0, x1, rots):
        for r in rots:
            x0 = x0 + x1
            x1 = (x1 << np.uint32(r)) | (x1 >> np.uint32(32 - r))
            x1 = x0 ^ x1
        return x0, x1

    x0 = x0 + ks0
    x1 = x1 + ks1
    x0, x1 = rounds(x0, x1, rot_a)
    x0 = x0 + ks1
    x1 = x1 + ks2 + np.uint32(1)
    x0, x1 = rounds(x0, x1, rot_b)
    x0 = x0 + ks2
    x1 = x1 + ks0 + np.uint32(2)
    x0, x1 = rounds(x0, x1, rot_a)
    x0 = x0 + ks0
    x1 = x1 + ks1 + np.uint32(3)
    x0, x1 = rounds(x0, x1, rot_b)
    x0 = x0 + ks1
    x1 = x1 + ks2 + np.uint32(4)
    x0, x1 = rounds(x0, x1, rot_a)
    x0 = x0 + ks2
    x1 = x1 + ks0 + np.uint32(5)
    return x0, x1


_z = np.zeros(2, np.uint32)
_k_lo, _k_hi = _threefry2x32(np.uint32(0), np.uint32(42), _z,
                             np.arange(2, dtype=np.uint32))
_K1 = (np.uint32(_k_lo[0]), np.uint32(_k_hi[0]))
_K2 = (np.uint32(_k_lo[1]), np.uint32(_k_hi[1]))


def _bits(key, x1):
    y0, y1 = _threefry2x32(key[0], key[1], jnp.zeros_like(x1), x1)
    return y0 ^ y1


def _flip_body(x_ref, o_ref):
    tile_rows = x_ref.shape[0]
    x = x_ref[...]
    row = jax.lax.broadcasted_iota(jnp.int32, (tile_rows, _D), 0)
    lane = jax.lax.broadcasted_iota(jnp.int32, (tile_rows, _D), 1)
    b = pl.program_id(0) * tile_rows + row
    col = jnp.minimum(lane, _NBIN - 1)

    cond_bits = _bits(_K2, (b * _NBIN + col).astype(jnp.uint32))
    u = jax.lax.bitcast_convert_type(
        (cond_bits >> np.uint32(9)) | np.uint32(_MASK23), jnp.float32
    ) - np.float32(1.0)
    keep = u >= _RATE

    f0 = (col * (2 * _B) + 2 * b).astype(jnp.uint32)
    m0 = _bits(_K1, f0) >> np.uint32(9)
    m1 = _bits(_K1, f0 + np.uint32(1)) >> np.uint32(9)
    samp = jnp.where(m1 > m0, np.float32(1.0), np.float32(0.0))

    flip = jnp.logical_and(lane < _NBIN, jnp.logical_not(keep))
    o_ref[...] = jnp.where(flip, samp, x)


def _flipped(inputs):
    tile_rows = 1024
    return pl.pallas_call(
        _flip_body,
        grid=(_B // tile_rows,),
        in_specs=[pl.BlockSpec((tile_rows, _D), lambda i: (i, 0))],
        out_specs=pl.BlockSpec((tile_rows, _D), lambda i: (i, 0)),
        out_shape=jax.ShapeDtypeStruct((_B, _D), jnp.float32),
    )(inputs)


def kernel(inputs, training):
    return jax.lax.cond(
        jnp.asarray(training) != 0, _flipped, lambda x: x, inputs
    )


# speedup vs baseline: 3.9618x; 1.0369x over previous
import numpy as np
import jax
import jax.numpy as jnp
from jax.experimental import pallas as pl

_B = 16384
_D = 128
_NBIN = 100
_RATE = np.float32(0.01)
_MASK23 = 0x3F800000


def _threefry2x32(ks0, ks1, x0, x1):
    ks2 = ks0 ^ ks1 ^ np.uint32(0x1BD11BDA)
    rot_a = (13, 15, 26, 6)
    rot_b = (17, 29, 16, 24)

    def rounds(x0, x1, rots):
        for r in rots:
            x0 = x0 + x1
            x1 = (x1 * np.uint32(1 << r)) | (x1 >> np.uint32(32 - r))
            x1 = x0 ^ x1
        return x0, x1

    x0 = x0 + ks0
    x1 = x1 + ks1
    x0, x1 = rounds(x0, x1, rot_a)
    x0 = x0 + ks1
    x1 = x1 + (ks2 + np.uint32(1))
    x0, x1 = rounds(x0, x1, rot_b)
    x0 = x0 + ks2
    x1 = x1 + (ks0 + np.uint32(2))
    x0, x1 = rounds(x0, x1, rot_a)
    x0 = x0 + ks0
    x1 = x1 + (ks1 + np.uint32(3))
    x0, x1 = rounds(x0, x1, rot_b)
    x0 = x0 + ks1
    x1 = x1 + (ks2 + np.uint32(4))
    x0, x1 = rounds(x0, x1, rot_a)
    x0 = x0 + ks2
    x1 = x1 + (ks0 + np.uint32(5))
    return x0, x1


_k_lo, _k_hi = _threefry2x32(np.uint32(0), np.uint32(42), np.uint32(0),
                             np.arange(2, dtype=np.uint32))
_K1 = (np.uint32(_k_lo[0]), np.uint32(_k_hi[0]))
_K2 = (np.uint32(_k_lo[1]), np.uint32(_k_hi[1]))


def _bits(key, x1):
    y0, y1 = _threefry2x32(key[0], key[1], np.uint32(0), x1)
    return y0 ^ y1


def _flip_body(x_ref, o_ref):
    tile_rows = x_ref.shape[0]
    x = x_ref[...]
    row = jax.lax.broadcasted_iota(jnp.int32, (tile_rows, _D), 0)
    lane = jax.lax.broadcasted_iota(jnp.int32, (tile_rows, _D), 1)
    b = pl.program_id(0) * tile_rows + row
    col = jnp.minimum(lane, _NBIN - 1)

    cond_bits = _bits(_K2, (b * _NBIN + col).astype(jnp.uint32))
    u = jax.lax.bitcast_convert_type(
        (cond_bits >> np.uint32(9)) | np.uint32(_MASK23), jnp.float32
    ) - np.float32(1.0)
    keep = u >= _RATE

    f0 = (col * (2 * _B) + 2 * b).astype(jnp.uint32)
    m0 = _bits(_K1, f0) >> np.uint32(9)
    m1 = _bits(_K1, f0 + np.uint32(1)) >> np.uint32(9)
    samp = jnp.where(m1 > m0, np.float32(1.0), np.float32(0.0))

    flip = jnp.logical_and(lane < _NBIN, jnp.logical_not(keep))
    o_ref[...] = jnp.where(flip, samp, x)


def _flipped(inputs):
    tile_rows = 1024
    return pl.pallas_call(
        _flip_body,
        grid=(_B // tile_rows,),
        in_specs=[pl.BlockSpec((tile_rows, _D), lambda i: (i, 0))],
        out_specs=pl.BlockSpec((tile_rows, _D), lambda i: (i, 0)),
        out_shape=jax.ShapeDtypeStruct((_B, _D), jnp.float32),
    )(inputs)


def kernel(inputs, training):
    return jax.lax.cond(
        jnp.asarray(training) != 0, _flipped, lambda x: x, inputs
    )
